# baseline (speedup 1.0000x reference)
import functools

import jax
import jax.numpy as jnp
from jax import lax
from jax.experimental import pallas as pl
from jax.experimental.pallas import tpu as pltpu
from jax.experimental.pallas import tpu_sc as plsc


def _embed_flat(idx, table, chunk):
    n, = idx.shape
    v, d = table.shape
    info = plsc.get_sparse_core_info()
    nw = info.num_cores * info.num_subcores
    per_w = n // nw
    assert per_w * nw == n and per_w % chunk == 0
    steps = per_w // chunk
    mesh = plsc.VectorSubcoreMesh(core_axis_name="c", subcore_axis_name="s")

    @functools.partial(
        pl.kernel,
        mesh=mesh,
        compiler_params=pltpu.CompilerParams(use_tc_tiling_on_sc=False),
        out_type=jax.ShapeDtypeStruct((n, d), jnp.float32),
        scratch_types=[
            pltpu.VMEM((chunk,), jnp.int32),
            pltpu.VMEM((chunk, d), jnp.float32),
            pltpu.SemaphoreType.DMA,
        ],
    )
    def emb(idx_hbm, table_hbm, out_hbm, idx_v, rows_v, sem):
        wid = lax.axis_index("s") * info.num_cores + lax.axis_index("c")
        base = wid * per_w

        def body(i, carry):
            off = base + i * chunk
            pltpu.sync_copy(idx_hbm.at[pl.ds(off, chunk)], idx_v)
            pltpu.async_copy(table_hbm.at[idx_v], rows_v, sem).wait()
            pltpu.sync_copy(rows_v, out_hbm.at[pl.ds(off, chunk)])
            return carry

        lax.fori_loop(0, steps, body, 0)

    return emb(idx, table)


def kernel(x, table):
    b, h = x.shape
    _, d = table.shape
    idx = x.reshape(b * h).astype(jnp.int32)
    out = _embed_flat(idx, table, chunk=128)
    return out.reshape(b, h, d)


# speedup vs baseline: 1.1204x; 1.1204x over previous
import functools

import jax
import jax.numpy as jnp
from jax import lax
from jax.experimental import pallas as pl
from jax.experimental.pallas import tpu as pltpu
from jax.experimental.pallas import tpu_sc as plsc


def _embed_flat(idx, table, chunk):
    n, = idx.shape
    v, d = table.shape
    info = plsc.get_sparse_core_info()
    nw = info.num_cores * info.num_subcores
    per_w = n // nw
    assert per_w * nw == n and per_w % chunk == 0
    steps = per_w // chunk
    mesh = plsc.VectorSubcoreMesh(core_axis_name="c", subcore_axis_name="s")

    @functools.partial(
        pl.kernel,
        mesh=mesh,
        compiler_params=pltpu.CompilerParams(use_tc_tiling_on_sc=False),
        out_type=jax.ShapeDtypeStruct((n, d), jnp.float32),
        scratch_types=[
            pltpu.VMEM((chunk,), jnp.int32),
            pltpu.VMEM((chunk, d), jnp.float32),
            pltpu.SemaphoreType.DMA,
        ],
    )
    def emb(idx_hbm, table_hbm, out_hbm, idx_v, rows_v, sem):
        wid = lax.axis_index("s") * info.num_cores + lax.axis_index("c")
        base = wid * per_w

        def body(i, carry):
            off = base + i * chunk
            pltpu.sync_copy(idx_hbm.at[pl.ds(off, chunk)], idx_v)
            pltpu.async_copy(table_hbm.at[idx_v], rows_v, sem).wait()
            pltpu.sync_copy(rows_v, out_hbm.at[pl.ds(off, chunk)])
            return carry

        lax.fori_loop(0, steps, body, 0)

    return emb(idx, table)


def kernel(x, table):
    b, h = x.shape
    _, d = table.shape
    idx = x.reshape(b * h).astype(jnp.int32)
    out = _embed_flat(idx, table, chunk=1024)
    return out.reshape(b, h, d)


# speedup vs baseline: 1.1385x; 1.0162x over previous
import functools

import jax
import jax.numpy as jnp
from jax import lax
from jax.experimental import pallas as pl
from jax.experimental.pallas import tpu as pltpu
from jax.experimental.pallas import tpu_sc as plsc


def _embed_flat(idx, table, chunk, nslot):
    n, = idx.shape
    _, d = table.shape
    info = plsc.get_sparse_core_info()
    nw = info.num_cores * info.num_subcores
    per_w = n // nw
    group = chunk * nslot
    assert per_w * nw == n and per_w % group == 0
    ngroups = per_w // group
    mesh = plsc.VectorSubcoreMesh(core_axis_name="c", subcore_axis_name="s")

    @functools.partial(
        pl.kernel,
        mesh=mesh,
        compiler_params=pltpu.CompilerParams(use_tc_tiling_on_sc=False),
        out_type=jax.ShapeDtypeStruct((n, d), jnp.float32),
        scratch_types=[
            pltpu.VMEM((per_w,), jnp.int32),
            *[pltpu.VMEM((chunk, d), jnp.float32) for _ in range(nslot)],
            *[pltpu.SemaphoreType.DMA for _ in range(2 * nslot)],
        ],
    )
    def emb(idx_hbm, table_hbm, out_hbm, idx_all, *refs):
        rows = refs[:nslot]
        gsem = refs[nslot:2 * nslot]
        ssem = refs[2 * nslot:]
        wid = lax.axis_index("s") * info.num_cores + lax.axis_index("c")
        base = wid * per_w
        pltpu.sync_copy(idx_hbm.at[pl.ds(base, per_w)], idx_all)

        def gather_start(i, s):
            pltpu.async_copy(
                table_hbm.at[idx_all.at[pl.ds(i * chunk, chunk)]],
                rows[s], gsem[s])

        def gather_wait(s):
            pltpu.make_async_copy(
                table_hbm.at[idx_all.at[pl.ds(0, chunk)]],
                rows[s], gsem[s]).wait()

        def store_wait(s):
            pltpu.make_async_copy(
                rows[s], out_hbm.at[pl.ds(base, chunk)], ssem[s]).wait()

        for s in range(nslot):
            gather_start(s, s)

        def body(g, carry):
            for s in range(nslot):
                i = g * nslot + s
                gather_wait(s)
                pltpu.async_copy(
                    rows[s], out_hbm.at[pl.ds(base + i * chunk, chunk)],
                    ssem[s])

            @pl.when(g + 1 < ngroups)
            def _():
                for s in range(nslot):
                    store_wait(s)
                    gather_start((g + 1) * nslot + s, s)

            return carry

        lax.fori_loop(0, ngroups, body, 0)
        for s in range(nslot):
            store_wait(s)

    return emb(idx, table)


def kernel(x, table):
    b, h = x.shape
    _, d = table.shape
    idx = x.reshape(b * h).astype(jnp.int32)
    out = _embed_flat(idx, table, chunk=320, nslot=8)
    return out.reshape(b, h, d)


# speedup vs baseline: 1.8382x; 1.6146x over previous
import functools

import jax
import jax.numpy as jnp
from jax import lax
from jax.experimental import pallas as pl
from jax.experimental.pallas import tpu as pltpu
from jax.experimental.pallas import tpu_sc as plsc


def kernel(x, table):
    b, h = x.shape
    _, d = table.shape
    idx = x.astype(jnp.int32)

    info = plsc.get_sparse_core_info()
    nw = info.num_cores * info.num_subcores
    b_per_w = b // nw
    bchunk = 8
    nslot = 8
    steps = b_per_w // bchunk
    ngroups = steps // nslot
    assert b_per_w * nw == b and steps * bchunk == b_per_w
    assert ngroups * nslot == steps
    mesh = plsc.VectorSubcoreMesh(core_axis_name="c", subcore_axis_name="s")

    @functools.partial(
        pl.kernel,
        mesh=mesh,
        compiler_params=pltpu.CompilerParams(use_tc_tiling_on_sc=False),
        out_type=jax.ShapeDtypeStruct((b, h, d), jnp.float32),
        scratch_types=[
            pltpu.VMEM((b_per_w, h), jnp.int32),
            *[pltpu.VMEM((bchunk, h, d), jnp.float32) for _ in range(nslot)],
            *[pltpu.SemaphoreType.DMA for _ in range(2 * nslot)],
        ],
    )
    def emb(idx_hbm, table_hbm, out_hbm, idx_all, *refs):
        rows = refs[:nslot]
        gsem = refs[nslot:2 * nslot]
        ssem = refs[2 * nslot:]
        wid = lax.axis_index("s") * info.num_cores + lax.axis_index("c")
        bbase = wid * b_per_w
        pltpu.sync_copy(idx_hbm.at[pl.ds(bbase, b_per_w)], idx_all)

        def out_slice(i):
            return out_hbm.at[pl.ds(bbase + i * bchunk, bchunk)]

        def gather_start(i, s):
            for j in range(bchunk):
                pltpu.async_copy(
                    table_hbm.at[idx_all.at[i * bchunk + j]],
                    rows[s].at[j], gsem[s])

        def gather_wait(s):
            pltpu.make_async_copy(out_slice(0), rows[s], gsem[s]).wait()

        def store_wait(s):
            pltpu.make_async_copy(rows[s], out_slice(0), ssem[s]).wait()

        for s in range(nslot):
            gather_start(s, s)

        def body(g, carry):
            for s in range(nslot):
                i = g * nslot + s
                gather_wait(s)
                pltpu.async_copy(rows[s], out_slice(i), ssem[s])

            @pl.when(g + 1 < ngroups)
            def _():
                for s in range(nslot):
                    store_wait(s)
                    gather_start((g + 1) * nslot + s, s)

            return carry

        lax.fori_loop(0, ngroups, body, 0)
        for s in range(nslot):
            store_wait(s)

    return emb(idx, table)
